# baseline (speedup 1.0000x reference)
import functools

import jax
import jax.numpy as jnp
from jax import lax
from jax.experimental import pallas as pl
from jax.experimental.pallas import tpu as pltpu
from jax.experimental.pallas import tpu_sc as plsc

B, T, ENC, FILT = 4, 4096, 1024, 1024
N = B * T
NBUCKETS = 256

NC, NS, L = 2, 16, 16
NW = NC * NS
ROWS_PER_W = N // NW
CHUNK = 32
NCHUNKS = ROWS_PER_W // CHUNK


def _sc_body(v_hbm, bk_hbm, emb_hbm, out_hbm,
             v_v, bk_v, idx_v, buf0, buf1, sem0, sem1):
  cid = lax.axis_index("c")
  sid = lax.axis_index("s")
  wid = sid * NC + cid
  base = wid * ROWS_PER_W

  pltpu.sync_copy(v_hbm.at[pl.ds(base, ROWS_PER_W)], v_v)
  pltpu.sync_copy(bk_hbm, bk_v)

  def search(r, carry):
    v = v_v[pl.ds(r * L, L)]
    idx = jnp.zeros((L,), jnp.int32)
    for step in (128, 64, 32, 16, 8, 4, 2, 1):
      cand = idx + step
      bval = plsc.load_gather(bk_v, [cand - 1])
      idx = jnp.where(bval < v, cand, idx)
    idx_v[pl.ds(r * L, L)] = jnp.minimum(idx, NBUCKETS - 1)
    return carry

  lax.fori_loop(0, ROWS_PER_W // L, search, 0)

  prev = None
  for c in range(NCHUNKS):
    buf, sem = (buf0, sem0) if c % 2 == 0 else (buf1, sem1)
    h = pltpu.async_copy(emb_hbm.at[idx_v.at[pl.ds(c * CHUNK, CHUNK)]],
                         buf, sem)
    if prev is not None:
      ph, pbuf, pc = prev
      ph.wait()
      pltpu.sync_copy(pbuf, out_hbm.at[pl.ds(base + pc * CHUNK, CHUNK)])
    prev = (h, buf, c)
  ph, pbuf, pc = prev
  ph.wait()
  pltpu.sync_copy(pbuf, out_hbm.at[pl.ds(base + pc * CHUNK, CHUNK)])


def _sc_gather(v, buckets, emb):
  run = functools.partial(
      pl.kernel,
      out_type=jax.ShapeDtypeStruct((N, ENC), jnp.float32),
      mesh=plsc.VectorSubcoreMesh(core_axis_name="c", subcore_axis_name="s",
                                  num_cores=NC, num_subcores=NS),
      compiler_params=pltpu.CompilerParams(needs_layout_passes=False),
      scratch_types=[
          pltpu.VMEM((ROWS_PER_W,), jnp.float32),
          pltpu.VMEM((NBUCKETS,), jnp.float32),
          pltpu.VMEM((ROWS_PER_W,), jnp.int32),
          pltpu.VMEM((CHUNK, ENC), jnp.float32),
          pltpu.VMEM((CHUNK, ENC), jnp.float32),
          pltpu.SemaphoreType.DMA,
          pltpu.SemaphoreType.DMA,
      ],
  )(_sc_body)
  return run(v, buckets, emb)


TR = 512
NT = N // TR
TPB = T // TR


def _ln(h, g, b):
  mu = jnp.mean(h, axis=-1, keepdims=True)
  var = jnp.mean((h - mu) ** 2, axis=-1, keepdims=True)
  return (h - mu) * lax.rsqrt(var + 1e-5) * g + b


def _dot(a, w):
  return lax.dot_general(a, w, (((1,), (0,)), ((), ())),
                         preferred_element_type=jnp.float32)


def _split(a):
  hi = a.astype(jnp.bfloat16)
  lo = (a - hi.astype(jnp.float32)).astype(jnp.bfloat16)
  return hi, lo


F8 = jnp.float8_e4m3fn
SA = 2.0 ** 3
SB = 2.0 ** 9
SWH = 2.0 ** 13
SWL = 2.0 ** 22


def _conv3c(xa, whi_ref, wlo8_ref, whi8_ref, rc):
  hi = xa.astype(jnp.bfloat16)
  x8 = (xa * SA).astype(F8)
  l8 = ((xa - hi.astype(jnp.float32)) * SB).astype(F8)
  cat = lambda t: jnp.concatenate([t[0:rc], t[1:rc + 1], t[2:rc + 2]], axis=1)
  return (_dot(cat(hi), whi_ref[...])
          + _dot(cat(x8), wlo8_ref[...]) * jnp.float32(1 / (SA * SWL))
          + _dot(cat(l8), whi8_ref[...]) * jnp.float32(1 / (SB * SWH)))


CHUNKS1 = [(0, 128), (128, 128), (256, 128), (384, TR + 2 - 384)]
CHUNKS2 = [(0, 128), (128, 128), (256, 128), (384, TR - 384)]

WPR = 1024


def _wprep_body(w1_ref, w2_ref, w1h_ref, w1l8_ref, w1h8_ref,
                w2h_ref, w2l8_ref, w2h8_ref):
  for src, dh, dl8, dh8 in ((w1_ref, w1h_ref, w1l8_ref, w1h8_ref),
                            (w2_ref, w2h_ref, w2l8_ref, w2h8_ref)):
    w = src[...]
    hi = w.astype(jnp.bfloat16)
    dh[...] = hi
    dl8[...] = ((w - hi.astype(jnp.float32)) * SWL).astype(F8)
    dh8[...] = (hi.astype(jnp.float32) * SWH).astype(F8)


def _wprep(w1r, w2r):
  blk = lambda: pl.BlockSpec((WPR, FILT), lambda i: (i, 0))
  nw = 3 * ENC // WPR
  sd = lambda dt: jax.ShapeDtypeStruct((3 * ENC, FILT), dt)
  return pl.pallas_call(
      _wprep_body,
      grid=(nw,),
      in_specs=[blk(), blk()],
      out_specs=[blk(), blk(), blk(), blk(), blk(), blk()],
      out_shape=[sd(jnp.bfloat16), sd(F8), sd(F8),
                 sd(jnp.bfloat16), sd(F8), sd(F8)],
  )(w1r, w2r)


def _tc_body(xc_ref, halo_ref, w1h_ref, w1l8_ref, w1h8_ref,
             w2h_ref, w2l8_ref, w2h8_ref,
             b1_ref, g1_ref, be1_ref, b2_ref, g2_ref, be2_ref,
             wl_ref, bl_ref, out_ref, h1s):
  ib = pl.program_id(0) % TPB

  def win1(r0, rc):
    lo, hi = r0 - 2, r0 + rc
    if lo < 0:
      return jnp.concatenate([halo_ref[0, 0:2, :], xc_ref[0:hi, :]], axis=0)
    if hi > TR:
      return jnp.concatenate([xc_ref[lo:TR, :], halo_ref[0, 2:4, :]], axis=0)
    return xc_ref[lo:hi, :]

  for r0, rc in CHUNKS1:
    acc = _conv3c(win1(r0, rc), w1h_ref, w1l8_ref, w1h8_ref, rc)
    h = jnp.maximum(acc + b1_ref[...], 0.0)
    h1n = _ln(h, g1_ref[...], be1_ref[...])
    row = lax.broadcasted_iota(jnp.int32, (rc, 1), 0) + r0
    edge = ((ib == 0) & (row == 0)) | ((ib == TPB - 1) & (row == TR + 1))
    h1s[r0:r0 + rc, :] = jnp.where(edge, 0.0, h1n)

  for r0, rc in CHUNKS2:
    acc2 = _conv3c(h1s[r0:r0 + rc + 2, :], w2h_ref, w2l8_ref, w2h8_ref, rc)
    h2 = jnp.maximum(acc2 + b2_ref[...], 0.0)
    h2n = _ln(h2, g2_ref[...], be2_ref[...])
    ans = jnp.sum(h2n * wl_ref[...], axis=-1) + bl_ref[0, 0]
    out_ref[0, 0, r0:r0 + rc] = ans


def _tc_predictor(xb, halos, w1h, w1l8, w1h8, w2h, w2l8, w2h8,
                  b1, g1, be1, b2, g2, be2, wl, bl):
  full = lambda shape: pl.BlockSpec(shape, lambda i: (0,) * len(shape))
  return pl.pallas_call(
      _tc_body,
      grid=(NT,),
      in_specs=[
          pl.BlockSpec((TR, ENC), lambda i: (i, 0)),
          pl.BlockSpec((1, 4, ENC), lambda i: (i, 0, 0)),
          full((3 * ENC, FILT)), full((3 * ENC, FILT)), full((3 * ENC, FILT)),
          full((3 * FILT, FILT)), full((3 * FILT, FILT)), full((3 * FILT, FILT)),
          full((1, FILT)), full((1, FILT)), full((1, FILT)),
          full((1, FILT)), full((1, FILT)), full((1, FILT)),
          full((1, FILT)),
          full((1, 1)),
      ],
      out_specs=pl.BlockSpec((1, 1, TR), lambda i: (i, 0, 0)),
      out_shape=jax.ShapeDtypeStruct((NT, 1, TR), jnp.float32),
      scratch_shapes=[
          pltpu.VMEM((TR + 2, FILT), jnp.float32),
      ],
  )(xb, halos, w1h, w1l8, w1h8, w2h, w2l8, w2h8,
    b1, g1, be1, b2, g2, be2, wl, bl)


def kernel(x, target, emb, W1, b1, g1, be1, W2, b2, g2, be2, Wl, bl, buckets):
  v = jnp.log1p(target).reshape(N)
  xb = x.reshape(N, ENC)
  z2 = jnp.zeros((B, 2, ENC), x.dtype)
  lefts = jnp.stack(
      [x[:, j * TR - 2:j * TR] if j > 0 else z2 for j in range(TPB)], 1)
  rights = jnp.stack(
      [x[:, (j + 1) * TR:(j + 1) * TR + 2] if j < TPB - 1 else z2
       for j in range(TPB)], 1)
  halos = jnp.concatenate([lefts, rights], 2).reshape(NT, 4, ENC)

  w1h, w1l8, w1h8, w2h, w2l8, w2h8 = _wprep(
      W1.reshape(3 * ENC, FILT), W2.reshape(3 * FILT, FILT))

  v, w1h = lax.optimization_barrier((v, w1h))
  out_emb = _sc_gather(v, buckets, emb)

  ans = _tc_predictor(
      xb, halos, w1h, w1l8, w1h8, w2h, w2l8, w2h8,
      b1.reshape(1, FILT), g1.reshape(1, FILT), be1.reshape(1, FILT),
      b2.reshape(1, FILT), g2.reshape(1, FILT), be2.reshape(1, FILT),
      Wl.reshape(1, FILT), bl.reshape(1, 1))

  return (out_emb.reshape(B, T, ENC), ans.reshape(B, T))


# speedup vs baseline: 1.0113x; 1.0113x over previous
import functools

import jax
import jax.numpy as jnp
from jax import lax
from jax.experimental import pallas as pl
from jax.experimental.pallas import tpu as pltpu
from jax.experimental.pallas import tpu_sc as plsc

B, T, ENC, FILT = 4, 4096, 1024, 1024
N = B * T
NBUCKETS = 256

NC, NS, L = 2, 16, 16
NW = NC * NS
ROWS_PER_W = N // NW
CHUNK = 32
NCHUNKS = ROWS_PER_W // CHUNK


def _sc_body(v_hbm, bk_hbm, emb_hbm, out_hbm,
             v_v, bk_v, idx_v, buf0, buf1, sem0, sem1):
  cid = lax.axis_index("c")
  sid = lax.axis_index("s")
  wid = sid * NC + cid
  base = wid * ROWS_PER_W

  pltpu.sync_copy(v_hbm.at[pl.ds(base, ROWS_PER_W)], v_v)
  pltpu.sync_copy(bk_hbm, bk_v)

  def search(r, carry):
    v = v_v[pl.ds(r * L, L)]
    idx = jnp.zeros((L,), jnp.int32)
    for step in (128, 64, 32, 16, 8, 4, 2, 1):
      cand = idx + step
      bval = plsc.load_gather(bk_v, [cand - 1])
      idx = jnp.where(bval < v, cand, idx)
    idx_v[pl.ds(r * L, L)] = jnp.minimum(idx, NBUCKETS - 1)
    return carry

  lax.fori_loop(0, ROWS_PER_W // L, search, 0)

  prev = None
  for c in range(NCHUNKS):
    buf, sem = (buf0, sem0) if c % 2 == 0 else (buf1, sem1)
    h = pltpu.async_copy(emb_hbm.at[idx_v.at[pl.ds(c * CHUNK, CHUNK)]],
                         buf, sem)
    if prev is not None:
      ph, pbuf, pc = prev
      ph.wait()
      pltpu.sync_copy(pbuf, out_hbm.at[pl.ds(base + pc * CHUNK, CHUNK)])
    prev = (h, buf, c)
  ph, pbuf, pc = prev
  ph.wait()
  pltpu.sync_copy(pbuf, out_hbm.at[pl.ds(base + pc * CHUNK, CHUNK)])


def _sc_gather(v, buckets, emb):
  run = functools.partial(
      pl.kernel,
      out_type=jax.ShapeDtypeStruct((N, ENC), jnp.float32),
      mesh=plsc.VectorSubcoreMesh(core_axis_name="c", subcore_axis_name="s",
                                  num_cores=NC, num_subcores=NS),
      compiler_params=pltpu.CompilerParams(needs_layout_passes=False),
      scratch_types=[
          pltpu.VMEM((ROWS_PER_W,), jnp.float32),
          pltpu.VMEM((NBUCKETS,), jnp.float32),
          pltpu.VMEM((ROWS_PER_W,), jnp.int32),
          pltpu.VMEM((CHUNK, ENC), jnp.float32),
          pltpu.VMEM((CHUNK, ENC), jnp.float32),
          pltpu.SemaphoreType.DMA,
          pltpu.SemaphoreType.DMA,
      ],
  )(_sc_body)
  return run(v, buckets, emb)


TR = 512
NT = N // TR
TPB = T // TR


def _ln(h, g, b):
  mu = jnp.mean(h, axis=-1, keepdims=True)
  var = jnp.mean((h - mu) ** 2, axis=-1, keepdims=True)
  return (h - mu) * lax.rsqrt(var + 1e-5) * g + b


def _dot(a, w):
  return lax.dot_general(a, w, (((1,), (0,)), ((), ())),
                         preferred_element_type=jnp.float32)


F8 = jnp.float8_e4m3fn
SA = 2.0 ** 3
SB = 2.0 ** 9
SWH = 2.0 ** 13
SWL = 2.0 ** 22


def _conv3c(xa, whi_ref, wlo8_ref, whi8_ref, rc):
  hi = xa.astype(jnp.bfloat16)
  x8 = (xa * SA).astype(F8)
  l8 = ((xa - hi.astype(jnp.float32)) * SB).astype(F8)
  cat = lambda t: jnp.concatenate([t[0:rc], t[1:rc + 1], t[2:rc + 2]], axis=1)
  return (_dot(cat(hi), whi_ref[...])
          + _dot(cat(x8), wlo8_ref[...]) * jnp.float32(1 / (SA * SWL))
          + _dot(cat(l8), whi8_ref[...]) * jnp.float32(1 / (SB * SWH)))


CHUNKS1 = [(0, 128), (128, 128), (256, 128), (384, TR + 2 - 384)]
CHUNKS2 = [(0, 128), (128, 128), (256, 128), (384, TR - 384)]

WPR = 1024


def _wprep_body(w1_ref, w2_ref, w1h_ref, w1l8_ref, w1h8_ref,
                w2h_ref, w2l8_ref, w2h8_ref):
  for src, dh, dl8, dh8 in ((w1_ref, w1h_ref, w1l8_ref, w1h8_ref),
                            (w2_ref, w2h_ref, w2l8_ref, w2h8_ref)):
    w = src[...]
    hi = w.astype(jnp.bfloat16)
    dh[...] = hi
    dl8[...] = ((w - hi.astype(jnp.float32)) * SWL).astype(F8)
    dh8[...] = (hi.astype(jnp.float32) * SWH).astype(F8)


def _wprep(w1r, w2r):
  blk = lambda: pl.BlockSpec((WPR, FILT), lambda i: (i, 0))
  nw = 3 * ENC // WPR
  sd = lambda dt: jax.ShapeDtypeStruct((3 * ENC, FILT), dt)
  return pl.pallas_call(
      _wprep_body,
      grid=(nw,),
      in_specs=[blk(), blk()],
      out_specs=[blk(), blk(), blk(), blk(), blk(), blk()],
      out_shape=[sd(jnp.bfloat16), sd(F8), sd(F8),
                 sd(jnp.bfloat16), sd(F8), sd(F8)],
  )(w1r, w2r)


def _tc_body(xc_ref, halo_ref, w1h_ref, w1l8_ref, w1h8_ref,
             w2h_ref, w2l8_ref, w2h8_ref,
             b1_ref, g1_ref, be1_ref, b2_ref, g2_ref, be2_ref,
             wl_ref, bl_ref, out_ref, xw, h1s):
  xw[0:2, :] = halo_ref[0, 0:2, :]
  xw[2:TR + 2, :] = xc_ref[...]
  xw[TR + 2:TR + 4, :] = halo_ref[0, 2:4, :]

  ib = pl.program_id(0) % TPB

  for r0, rc in CHUNKS1:
    acc = _conv3c(xw[r0:r0 + rc + 2, :], w1h_ref, w1l8_ref, w1h8_ref, rc)
    h = jnp.maximum(acc + b1_ref[...], 0.0)
    h1n = _ln(h, g1_ref[...], be1_ref[...])
    row = lax.broadcasted_iota(jnp.int32, (rc, 1), 0) + r0
    edge = ((ib == 0) & (row == 0)) | ((ib == TPB - 1) & (row == TR + 1))
    h1s[r0:r0 + rc, :] = jnp.where(edge, 0.0, h1n)

  for r0, rc in CHUNKS2:
    acc2 = _conv3c(h1s[r0:r0 + rc + 2, :], w2h_ref, w2l8_ref, w2h8_ref, rc)
    h2 = jnp.maximum(acc2 + b2_ref[...], 0.0)
    h2n = _ln(h2, g2_ref[...], be2_ref[...])
    ans = jnp.sum(h2n * wl_ref[...], axis=-1) + bl_ref[0, 0]
    out_ref[0, 0, r0:r0 + rc] = ans


def _tc_predictor(xb, halos, w1h, w1l8, w1h8, w2h, w2l8, w2h8,
                  b1, g1, be1, b2, g2, be2, wl, bl):
  full = lambda shape: pl.BlockSpec(shape, lambda i: (0,) * len(shape))
  return pl.pallas_call(
      _tc_body,
      grid=(NT,),
      in_specs=[
          pl.BlockSpec((TR, ENC), lambda i: (i, 0)),
          pl.BlockSpec((1, 4, ENC), lambda i: (i, 0, 0)),
          full((3 * ENC, FILT)), full((3 * ENC, FILT)), full((3 * ENC, FILT)),
          full((3 * FILT, FILT)), full((3 * FILT, FILT)), full((3 * FILT, FILT)),
          full((1, FILT)), full((1, FILT)), full((1, FILT)),
          full((1, FILT)), full((1, FILT)), full((1, FILT)),
          full((1, FILT)),
          full((1, 1)),
      ],
      out_specs=pl.BlockSpec((1, 1, TR), lambda i: (i, 0, 0)),
      out_shape=jax.ShapeDtypeStruct((NT, 1, TR), jnp.float32),
      scratch_shapes=[
          pltpu.VMEM((TR + 4, ENC), jnp.float32),
          pltpu.VMEM((TR + 2, FILT), jnp.float32),
      ],
  )(xb, halos, w1h, w1l8, w1h8, w2h, w2l8, w2h8,
    b1, g1, be1, b2, g2, be2, wl, bl)


def kernel(x, target, emb, W1, b1, g1, be1, W2, b2, g2, be2, Wl, bl, buckets):
  v = jnp.log1p(target).reshape(N)
  xb = x.reshape(N, ENC)
  z2 = jnp.zeros((B, 2, ENC), x.dtype)
  lefts = jnp.stack(
      [x[:, j * TR - 2:j * TR] if j > 0 else z2 for j in range(TPB)], 1)
  rights = jnp.stack(
      [x[:, (j + 1) * TR:(j + 1) * TR + 2] if j < TPB - 1 else z2
       for j in range(TPB)], 1)
  halos = jnp.concatenate([lefts, rights], 2).reshape(NT, 4, ENC)

  w1h, w1l8, w1h8, w2h, w2l8, w2h8 = _wprep(
      W1.reshape(3 * ENC, FILT), W2.reshape(3 * FILT, FILT))

  v, w1h = lax.optimization_barrier((v, w1h))
  out_emb = _sc_gather(v, buckets, emb)

  ans = _tc_predictor(
      xb, halos, w1h, w1l8, w1h8, w2h, w2l8, w2h8,
      b1.reshape(1, FILT), g1.reshape(1, FILT), be1.reshape(1, FILT),
      b2.reshape(1, FILT), g2.reshape(1, FILT), be2.reshape(1, FILT),
      Wl.reshape(1, FILT), bl.reshape(1, 1))

  return (out_emb.reshape(B, T, ENC), ans.reshape(B, T))
